# baseline (speedup 1.0000x reference)
import functools

import jax
import jax.numpy as jnp
from jax import lax
from jax.experimental import pallas as pl
from jax.experimental.pallas import tpu as pltpu
from jax.experimental.pallas import tpu_sc as plsc

NC = 2
NS = 16
NW = NC * NS
CHUNK = 128
NBUF = 8

PW = 16384
NPACK = 31


def _pack_table(tab_t):
    d, v = tab_t.shape

    def body(x_ref, o_ref):
        x = x_ref[...]
        o_ref[...] = jnp.concatenate([x[:, 0:PW], x[:, PW:2 * PW]], axis=0).T

    return pl.pallas_call(
        body,
        grid=(NPACK,),
        in_specs=[pl.BlockSpec((d, 2 * PW), lambda i: (0, i))],
        out_specs=pl.BlockSpec((PW, 2 * d), lambda i: (i, 0)),
        out_shape=jax.ShapeDtypeStruct((NPACK * PW, 2 * d), jnp.float32),
        compiler_params=pltpu.CompilerParams(
            dimension_semantics=("arbitrary",)),
    )(tab_t)


def _make_sc_gather(dim, n_chunks):
    mesh = plsc.VectorSubcoreMesh(core_axis_name="c", subcore_axis_name="s")
    total = NW * n_chunks * CHUNK

    @functools.partial(
        pl.kernel,
        mesh=mesh,
        out_type=jax.ShapeDtypeStruct((total // 2, 2 * dim), jnp.float32),
        compiler_params=pltpu.CompilerParams(use_tc_tiling_on_sc=False),
        scratch_types=(
            [pltpu.VMEM((n_chunks, CHUNK), jnp.int32)]
            + [pltpu.VMEM((CHUNK, dim), jnp.float32) for _ in range(NBUF)]
            + [pltpu.SemaphoreType.DMA for _ in range(2 * NBUF)]
        ),
    )
    def k(table_hbm, idx_hbm, out_hbm, idx_v, *bufs_and_sems):
        rows = bufs_and_sems[:NBUF]
        gsem = bufs_and_sems[NBUF:2 * NBUF]
        psem = bufs_and_sems[2 * NBUF:]
        wid = lax.axis_index("s") * NC + lax.axis_index("c")
        base = (wid & 15) * (n_chunks * CHUNK)
        col = dim * (wid >> 4)

        pltpu.sync_copy(idx_hbm.at[wid], idx_v)

        def gather(j, b):
            return pltpu.make_async_copy(
                table_hbm.at[idx_v.at[j]], rows[b], gsem[b])

        def put(j, b):
            return pltpu.make_async_copy(
                rows[b],
                out_hbm.at[pl.ds(base + j * CHUNK, CHUNK), pl.ds(col, dim)],
                psem[b])

        for b in range(NBUF):
            gather(b, b).start()

        def outer(g, _):
            for b in range(NBUF):
                j = g * NBUF + b
                gather(j, b).wait()
                put(j, b).start()
                put(j, b).wait()
                gather(j + NBUF, b).start()
            return _

        n_outer = n_chunks // NBUF
        lax.fori_loop(0, n_outer - 1, outer, None)

        for b in range(NBUF):
            j = (n_outer - 1) * NBUF + b
            gather(j, b).wait()
            put(j, b).start()
            put(j, b).wait()

    return k


def _transpose_out(rows3, n_fields, batch, dim):
    bb = 8192
    grid_b = batch // bb
    fh = n_fields // 2

    def body(x_ref, o_ref):
        h = pl.program_id(2)
        x = x_ref[0]

        @pl.when(h == 0)
        def _():
            o_ref[0] = x[:, 0:dim].T

        @pl.when(h == 1)
        def _():
            o_ref[0] = x[:, dim:2 * dim].T

    return pl.pallas_call(
        body,
        grid=(fh, grid_b, 2),
        in_specs=[pl.BlockSpec((1, bb, 2 * dim), lambda g, j, h: (g, j, 0))],
        out_specs=pl.BlockSpec((1, dim, bb), lambda g, j, h: (g + fh * h, 0, j)),
        out_shape=jax.ShapeDtypeStruct((n_fields, dim, batch), jnp.float32),
        compiler_params=pltpu.CompilerParams(
            dimension_semantics=("arbitrary", "arbitrary", "arbitrary")),
    )(rows3)


def kernel(tokenIndex, e_weights):
    batch, n_fields = tokenIndex.shape
    vocab, dim = e_weights.shape
    total = batch * n_fields
    n_chunks = total // (NW * CHUNK)

    tab_t = e_weights.T
    idx_flat = tokenIndex.T.reshape(-1)

    tab_packed = _pack_table(tab_t)
    tab_rm = tab_packed.reshape(2 * NPACK * PW, dim)

    idx_lin = ((idx_flat & ~(2 * PW - 1))
               + 2 * (idx_flat & (PW - 1))
               + ((idx_flat >> 14) & 1))
    idx3 = idx_lin.reshape(NW, n_chunks, CHUNK)

    rows = _make_sc_gather(dim, n_chunks)(tab_rm, idx3)

    rows3 = rows.reshape(n_fields // 2, batch, 2 * dim)
    out3 = _transpose_out(rows3, n_fields, batch, dim)
    return out3.transpose(2, 0, 1)


# speedup vs baseline: 1.0329x; 1.0329x over previous
import functools

import jax
import jax.numpy as jnp
from jax import lax
from jax.experimental import pallas as pl
from jax.experimental.pallas import tpu as pltpu
from jax.experimental.pallas import tpu_sc as plsc

NC = 2
NS = 16
NW = NC * NS
CHUNK = 128
NBUF = 13

PW = 16384
NPACK = 31


def _pack_table(tab_t):
    d, v = tab_t.shape

    def body(x_ref, o_ref):
        x = x_ref[...]
        o_ref[...] = jnp.concatenate([x[:, 0:PW], x[:, PW:2 * PW]], axis=0).T

    return pl.pallas_call(
        body,
        grid=(NPACK,),
        in_specs=[pl.BlockSpec((d, 2 * PW), lambda i: (0, i))],
        out_specs=pl.BlockSpec((PW, 2 * d), lambda i: (i, 0)),
        out_shape=jax.ShapeDtypeStruct((NPACK * PW, 2 * d), jnp.float32),
        compiler_params=pltpu.CompilerParams(
            dimension_semantics=("arbitrary",)),
    )(tab_t)


def _make_sc_gather(dim, n_chunks):
    mesh = plsc.VectorSubcoreMesh(core_axis_name="c", subcore_axis_name="s")
    total = NW * n_chunks * CHUNK

    @functools.partial(
        pl.kernel,
        mesh=mesh,
        out_type=jax.ShapeDtypeStruct((total // 2, 2 * dim), jnp.float32),
        compiler_params=pltpu.CompilerParams(use_tc_tiling_on_sc=False),
        scratch_types=(
            [pltpu.VMEM((n_chunks, CHUNK), jnp.int32)]
            + [pltpu.VMEM((CHUNK, dim), jnp.float32) for _ in range(NBUF)]
            + [pltpu.SemaphoreType.DMA for _ in range(2 * NBUF)]
        ),
    )
    def k(table_hbm, idx_hbm, out_hbm, idx_v, *bufs_and_sems):
        rows = bufs_and_sems[:NBUF]
        gsem = bufs_and_sems[NBUF:2 * NBUF]
        psem = bufs_and_sems[2 * NBUF:]
        wid = lax.axis_index("s") * NC + lax.axis_index("c")
        base = (wid & 15) * (n_chunks * CHUNK)
        col = dim * (wid >> 4)

        pltpu.sync_copy(idx_hbm.at[wid], idx_v)

        def gather(j, b):
            return pltpu.make_async_copy(
                table_hbm.at[idx_v.at[j]], rows[b], gsem[b])

        def put(j, b):
            return pltpu.make_async_copy(
                rows[b],
                out_hbm.at[pl.ds(base + j * CHUNK, CHUNK), pl.ds(col, dim)],
                psem[b])

        for b in range(NBUF):
            gather(b, b).start()

        def outer(g, _):
            for b in range(NBUF):
                j = g * NBUF + b
                gather(j, b).wait()
                put(j, b).start()
                put(j, b).wait()
                gather(j + NBUF, b).start()
            return _

        n_outer = n_chunks // NBUF
        lax.fori_loop(0, n_outer - 1, outer, None)

        for b in range(NBUF):
            j = (n_outer - 1) * NBUF + b
            gather(j, b).wait()
            put(j, b).start()
            put(j, b).wait()

    return k


def _transpose_out(rows3, n_fields, batch, dim):
    bb = 16384
    grid_b = batch // bb
    fh = n_fields // 2

    def body(x_ref, o_ref):
        h = pl.program_id(2)
        x = x_ref[0]

        @pl.when(h == 0)
        def _():
            o_ref[0] = x[:, 0:dim].T

        @pl.when(h == 1)
        def _():
            o_ref[0] = x[:, dim:2 * dim].T

    return pl.pallas_call(
        body,
        grid=(fh, grid_b, 2),
        in_specs=[pl.BlockSpec((1, bb, 2 * dim), lambda g, j, h: (g, j, 0))],
        out_specs=pl.BlockSpec((1, dim, bb), lambda g, j, h: (g + fh * h, 0, j)),
        out_shape=jax.ShapeDtypeStruct((n_fields, dim, batch), jnp.float32),
        compiler_params=pltpu.CompilerParams(
            dimension_semantics=("arbitrary", "arbitrary", "arbitrary")),
    )(rows3)


def kernel(tokenIndex, e_weights):
    batch, n_fields = tokenIndex.shape
    vocab, dim = e_weights.shape
    total = batch * n_fields
    n_chunks = total // (NW * CHUNK)

    tab_t = e_weights.T
    idx_flat = tokenIndex.T.reshape(-1)

    tab_packed = _pack_table(tab_t)
    tab_rm = tab_packed.reshape(2 * NPACK * PW, dim)

    idx_lin = ((idx_flat & ~(2 * PW - 1))
               + 2 * (idx_flat & (PW - 1))
               + ((idx_flat >> 14) & 1))
    idx3 = idx_lin.reshape(NW, n_chunks, CHUNK)

    rows = _make_sc_gather(dim, n_chunks)(tab_rm, idx3)

    rows3 = rows.reshape(n_fields // 2, batch, 2 * dim)
    out3 = _transpose_out(rows3, n_fields, batch, dim)
    return out3.transpose(2, 0, 1)


# speedup vs baseline: 1.0343x; 1.0014x over previous
import functools

import jax
import jax.numpy as jnp
from jax import lax
from jax.experimental import pallas as pl
from jax.experimental.pallas import tpu as pltpu
from jax.experimental.pallas import tpu_sc as plsc

NC = 2
NS = 16
NW = NC * NS
CHUNK = 128
NBUF = 8

PW = 16384
NPACK = 31


def _pack_table(tab_t):
    d, v = tab_t.shape

    def body(x_ref, o_ref):
        x = x_ref[...]
        o_ref[...] = jnp.concatenate([x[:, 0:PW], x[:, PW:2 * PW]], axis=0).T

    return pl.pallas_call(
        body,
        grid=(NPACK,),
        in_specs=[pl.BlockSpec((d, 2 * PW), lambda i: (0, i))],
        out_specs=pl.BlockSpec((PW, 2 * d), lambda i: (i, 0)),
        out_shape=jax.ShapeDtypeStruct((NPACK * PW, 2 * d), jnp.float32),
        compiler_params=pltpu.CompilerParams(
            dimension_semantics=("arbitrary",)),
    )(tab_t)


def _make_sc_gather(dim, n_chunks):
    mesh = plsc.VectorSubcoreMesh(core_axis_name="c", subcore_axis_name="s")
    total = NW * n_chunks * CHUNK

    @functools.partial(
        pl.kernel,
        mesh=mesh,
        out_type=jax.ShapeDtypeStruct((total // 2, 2 * dim), jnp.float32),
        compiler_params=pltpu.CompilerParams(use_tc_tiling_on_sc=False),
        scratch_types=(
            [pltpu.VMEM((n_chunks, CHUNK), jnp.int32)]
            + [pltpu.VMEM((CHUNK, dim), jnp.float32) for _ in range(NBUF)]
            + [pltpu.SemaphoreType.DMA for _ in range(2 * NBUF)]
        ),
    )
    def k(table_hbm, idx_hbm, out_hbm, idx_v, *bufs_and_sems):
        rows = bufs_and_sems[:NBUF]
        gsem = bufs_and_sems[NBUF:2 * NBUF]
        psem = bufs_and_sems[2 * NBUF:]
        wid = lax.axis_index("s") * NC + lax.axis_index("c")
        base = (wid & 15) * (n_chunks * CHUNK)
        col = dim * (wid >> 4)

        pltpu.sync_copy(idx_hbm.at[wid], idx_v)

        def gather(j, b):
            return pltpu.make_async_copy(
                table_hbm.at[idx_v.at[j]], rows[b], gsem[b])

        def put(j, b):
            return pltpu.make_async_copy(
                rows[b],
                out_hbm.at[pl.ds(base + j * CHUNK, CHUNK), pl.ds(col, dim)],
                psem[b])

        for b in range(NBUF):
            gather(b, b).start()

        def outer(g, _):
            for b in range(NBUF):
                j = g * NBUF + b
                gather(j, b).wait()
                put(j, b).start()
                put(j, b).wait()
                gather(j + NBUF, b).start()
            return _

        n_outer = n_chunks // NBUF
        lax.fori_loop(0, n_outer - 1, outer, None)

        for b in range(NBUF):
            j = (n_outer - 1) * NBUF + b
            gather(j, b).wait()
            put(j, b).start()
            put(j, b).wait()

    return k


def _transpose_out(rows3, n_fields, batch, dim):
    bb = 16384
    grid_b = batch // bb
    fh = n_fields // 2

    def body(x_ref, o_ref):
        h = pl.program_id(2)
        x = x_ref[0]

        @pl.when(h == 0)
        def _():
            o_ref[0] = x[:, 0:dim].T

        @pl.when(h == 1)
        def _():
            o_ref[0] = x[:, dim:2 * dim].T

    return pl.pallas_call(
        body,
        grid=(fh, grid_b, 2),
        in_specs=[pl.BlockSpec((1, bb, 2 * dim), lambda g, j, h: (g, j, 0))],
        out_specs=pl.BlockSpec((1, dim, bb), lambda g, j, h: (g + fh * h, 0, j)),
        out_shape=jax.ShapeDtypeStruct((n_fields, dim, batch), jnp.float32),
        compiler_params=pltpu.CompilerParams(
            dimension_semantics=("arbitrary", "arbitrary", "arbitrary")),
    )(rows3)


def kernel(tokenIndex, e_weights):
    batch, n_fields = tokenIndex.shape
    vocab, dim = e_weights.shape
    total = batch * n_fields
    n_chunks = total // (NW * CHUNK)

    tab_t = e_weights.T
    idx_flat = tokenIndex.T.reshape(-1)

    tab_packed = _pack_table(tab_t)
    tab_rm = tab_packed.reshape(2 * NPACK * PW, dim)

    idx_lin = ((idx_flat & ~(2 * PW - 1))
               + 2 * (idx_flat & (PW - 1))
               + ((idx_flat >> 14) & 1))
    idx3 = idx_lin.reshape(NW, n_chunks, CHUNK)

    rows = _make_sc_gather(dim, n_chunks)(tab_rm, idx3)

    rows3 = rows.reshape(n_fields // 2, batch, 2 * dim)
    out3 = _transpose_out(rows3, n_fields, batch, dim)
    return out3.transpose(2, 0, 1)
